# baseline (speedup 1.0000x reference)
import functools

import jax
import jax.numpy as jnp
from jax import lax
from jax.experimental import pallas as pl
from jax.experimental.pallas import tpu as pltpu
from jax.experimental.pallas import tpu_sc as plsc

_N = 10000
_E = 320000
_H = 64
_DE = 16

_NC = 2
_NS = 16
_NW = _NC * _NS
_EW = _E // _NW
_B = 80
_NB = _EW // _B
_NPAD = 10240
_RPT = _NPAD // _NS

_mesh = plsc.VectorSubcoreMesh(core_axis_name="c", subcore_axis_name="s")


def _vbcast(v, idx):
    return lax.gather(
        v, idx[:, None],
        dimension_numbers=lax.GatherDimensionNumbers(
            offset_dims=(), collapsed_slice_dims=(0,), start_index_map=(0,)),
        slice_sizes=(1,),
        mode=lax.GatherScatterMode.PROMISE_IN_BOUNDS)


def _gat_pass_body(with_ea, W, src_hbm, dst_hbm, als_hbm, ald_hbm, ale_hbm,
                   ea_hbm, xs_hbm, out_hbm, acc, als_v, ald_v, srcv, dstv,
                   aev, eav, rows_v, payload_v, exv, sem):
    c = lax.axis_index("c")
    s = lax.axis_index("s")
    wid = s * _NC + c
    lane = lax.iota(jnp.int32, 16)
    zero16 = jnp.zeros((16,), jnp.float32)
    sel0 = jnp.where(lane == 0, 1.0, 0.0)
    sel1 = jnp.where(lane == 1, 1.0, 0.0)
    nch = W // 16

    pltpu.sync_copy(als_hbm, als_v)
    pltpu.sync_copy(ald_hbm, ald_v)

    def zrow(r, carry):
        for k in range(nch):
            payload_v[r, pl.ds(k * 16, 16)] = zero16
        return carry
    lax.fori_loop(0, _B, zrow, 0)

    base = s * _RPT
    for i in range(_RPT // _B):
        pltpu.sync_copy(payload_v, acc.at[pl.ds(base + i * _B, _B)])
    plsc.subcore_barrier()

    def block(b, carry):
        eb = wid * _EW + b * _B
        cps = [pltpu.async_copy(src_hbm.at[pl.ds(eb, _B)], srcv, sem),
               pltpu.async_copy(dst_hbm.at[pl.ds(eb, _B)], dstv, sem),
               pltpu.async_copy(ale_hbm.at[pl.ds(eb, _B)], aev, sem)]
        if with_ea:
            cps.append(pltpu.async_copy(ea_hbm.at[pl.ds(eb, _B)], eav, sem))
        for d in cps:
            d.wait()
        cp = pltpu.async_copy(xs_hbm.at[srcv], rows_v, sem)
        for g in range(_B // 16):
            s16 = srcv[pl.ds(g * 16, 16)]
            d16 = dstv[pl.ds(g * 16, 16)]
            a = (plsc.load_gather(als_v, [s16]) +
                 plsc.load_gather(ald_v, [d16]) +
                 aev[pl.ds(g * 16, 16)])
            a = jnp.maximum(a, a * 0.2)
            exv[pl.ds(g * 16, 16)] = jnp.exp(a)
        cp.wait()

        for g in range(_B // 16):
            exg = exv[pl.ds(g * 16, 16)]

            def row(r2, carry2):
                r = g * 16 + r2
                w = _vbcast(exg, jnp.full((16,), r2, jnp.int32))
                for k in range(4):
                    payload_v[r, pl.ds(k * 16, 16)] = (
                        rows_v[r, pl.ds(k * 16, 16)] * w)
                if with_ea:
                    payload_v[r, pl.ds(64, 16)] = eav[r, :]
                    payload_v[r, pl.ds(80, 16)] = w * sel0 + sel1
                else:
                    payload_v[r, pl.ds(64, 16)] = w * sel0
                return carry2
            lax.fori_loop(0, 16, row, 0)
        pltpu.sync_copy(payload_v, acc.at[dstv], add=True)
        return carry
    lax.fori_loop(0, _NB, block, 0)
    plsc.subcore_barrier()
    pltpu.sync_copy(acc.at[pl.ds(base, _RPT)], out_hbm.at[c, pl.ds(base, _RPT)])


def _make_gat_pass(with_ea):
    W = 96 if with_ea else 80
    return pl.kernel(
        functools.partial(_gat_pass_body, with_ea, W),
        out_type=jax.ShapeDtypeStruct((_NC, _NPAD, W), jnp.float32),
        mesh=_mesh,
        compiler_params=pltpu.CompilerParams(needs_layout_passes=False, use_tc_tiling_on_sc=False),
        scratch_types=[
            pltpu.VMEM_SHARED((_NPAD, W), jnp.float32),
            pltpu.VMEM((_N,), jnp.float32),
            pltpu.VMEM((_N,), jnp.float32),
            pltpu.VMEM((_B,), jnp.int32),
            pltpu.VMEM((_B,), jnp.int32),
            pltpu.VMEM((_B,), jnp.float32),
            pltpu.VMEM((_B, _DE), jnp.float32),
            pltpu.VMEM((_B, _H), jnp.float32),
            pltpu.VMEM((_B, W), jnp.float32),
            pltpu.VMEM((_B,), jnp.float32),
            pltpu.SemaphoreType.DMA,
        ],
    )


_gat_pass1 = _make_gat_pass(True)
_gat_pass2 = _make_gat_pass(False)


def _scorer_body(src_hbm, dst_hbm, p_hbm, q_hbm, ws2_hbm, bs2_hbm,
                 scores_hbm, srcv, dstv, prow, qrow, wsb, ws2v, bs2v,
                 scorev, semp, semq):
    c = lax.axis_index("c")
    s = lax.axis_index("s")
    wid = s * _NC + c
    lane = lax.iota(jnp.int32, 16)

    pltpu.sync_copy(ws2_hbm, ws2v)
    pltpu.sync_copy(bs2_hbm, bs2v)

    def bw(k, carry):
        colk = (jnp.full((16,), k, jnp.int32) + lane) & 63
        wsb[k, :] = plsc.load_gather(ws2v, [colk])
        return carry
    lax.fori_loop(0, _H, bw, 0)
    bias = bs2v[...]

    def block(b, carry):
        eb = wid * _EW + b * _B
        cp1 = pltpu.async_copy(src_hbm.at[pl.ds(eb, _B)], srcv, semp)
        cp2 = pltpu.async_copy(dst_hbm.at[pl.ds(eb, _B)], dstv, semp)
        cp1.wait()
        cp2.wait()
        cpp = pltpu.async_copy(p_hbm.at[srcv], prow, semp)
        cpq = pltpu.async_copy(q_hbm.at[dstv], qrow, semq)
        cpp.wait()
        cpq.wait()
        for g in range(_B // 16):
            rvec = g * 16 + lane
            acc = bias
            for k in range(_H):
                colk = (jnp.full((16,), k, jnp.int32) + lane) & 63
                pk = plsc.load_gather(prow, [rvec, colk])
                qk = plsc.load_gather(qrow, [rvec, colk])
                acc = acc + jnp.maximum(pk + qk, 0.0) * wsb[k, :]
            scorev[pl.ds(g * 16, 16)] = 1.0 / (1.0 + jnp.exp(-acc))
        pltpu.sync_copy(scorev, scores_hbm.at[pl.ds(eb, _B)])
        return carry
    lax.fori_loop(0, _NB, block, 0)


_scorer = pl.kernel(
    _scorer_body,
    out_type=jax.ShapeDtypeStruct((_E,), jnp.float32),
    mesh=_mesh,
    compiler_params=pltpu.CompilerParams(needs_layout_passes=False, use_tc_tiling_on_sc=False),
    scratch_types=[
        pltpu.VMEM((_B,), jnp.int32),
        pltpu.VMEM((_B,), jnp.int32),
        pltpu.VMEM((_B, _H), jnp.float32),
        pltpu.VMEM((_B, _H), jnp.float32),
        pltpu.VMEM((_H, 16), jnp.float32),
        pltpu.VMEM((_H,), jnp.float32),
        pltpu.VMEM((16,), jnp.float32),
        pltpu.VMEM((_B,), jnp.float32),
        pltpu.SemaphoreType.DMA,
        pltpu.SemaphoreType.DMA,
    ],
)


_EB = 8000


def _leaky(x):
    return jnp.maximum(x, 0.2 * x)


def _proj1_body(x_ref, W1_ref, asd1_ref, Wee_ref, bee_ref, We1_ref, ae1_ref,
                We2_ref, ae2_ref, xs_ref, asd_ref, v12_ref, c12_ref):
    w1e = jnp.dot(We1_ref[...], ae1_ref[...], preferred_element_type=jnp.float32)
    w2e = jnp.dot(We2_ref[...], ae2_ref[...], preferred_element_type=jnp.float32)
    wboth = jnp.concatenate([w1e, w2e], axis=1)
    v12_ref[...] = jnp.dot(Wee_ref[...], wboth, preferred_element_type=jnp.float32)
    c12_ref[...] = jnp.dot(bee_ref[...], wboth, preferred_element_type=jnp.float32)
    xs = jnp.dot(x_ref[...], W1_ref[...], preferred_element_type=jnp.float32)
    xs_ref[...] = xs
    asd_ref[...] = jnp.dot(xs, asd1_ref[...], preferred_element_type=jnp.float32)


_NBLK = 1000


def _proj1(x, W1, asd1, Wee, bee, We1, ae1, We2, ae2):
    zero = lambda i: (0, 0)
    return pl.pallas_call(
        _proj1_body,
        grid=(_N // _NBLK,),
        in_specs=[
            pl.BlockSpec((_NBLK, 128), lambda i: (i, 0)),
            pl.BlockSpec((128, _H), zero),
            pl.BlockSpec((_H, 2), zero),
            pl.BlockSpec((_DE, _H), zero),
            pl.BlockSpec((1, _H), zero),
            pl.BlockSpec((_H, _H), zero),
            pl.BlockSpec((_H, 1), zero),
            pl.BlockSpec((_H, _H), zero),
            pl.BlockSpec((_H, 1), zero),
        ],
        out_specs=(
            pl.BlockSpec((_NBLK, _H), lambda i: (i, 0)),
            pl.BlockSpec((_NBLK, 2), lambda i: (i, 0)),
            pl.BlockSpec((_DE, 2), zero),
            pl.BlockSpec((1, 2), zero),
        ),
        out_shape=(
            jax.ShapeDtypeStruct((_N, _H), jnp.float32),
            jax.ShapeDtypeStruct((_N, 2), jnp.float32),
            jax.ShapeDtypeStruct((_DE, 2), jnp.float32),
            jax.ShapeDtypeStruct((1, 2), jnp.float32),
        ),
    )(x, W1, asd1, Wee, bee, We1, ae1, We2, ae2)


def _elog_body(ea_ref, v_ref, c_ref, out_ref):
    out_ref[...] = jnp.dot(ea_ref[...], v_ref[...],
                           preferred_element_type=jnp.float32) + c_ref[...]


def _edge_logits(edge_attr, v12, c12):
    return pl.pallas_call(
        _elog_body,
        grid=(_E // _EB,),
        in_specs=[
            pl.BlockSpec((_EB, _DE), lambda i: (i, 0)),
            pl.BlockSpec((_DE, 2), lambda i: (0, 0)),
            pl.BlockSpec((1, 2), lambda i: (0, 0)),
        ],
        out_specs=pl.BlockSpec((_EB, 2), lambda i: (i, 0)),
        out_shape=jax.ShapeDtypeStruct((_E, 2), jnp.float32),
    )(edge_attr, v12, c12)


def _fold1_body(o1a_ref, o1b_ref, xs1_ref, asd1v_ref, v12_ref, c12_ref,
                W2_ref, asd2w_ref, b1_ref, xs2_ref, asd2_ref, exl2_ref,
                mean16_ref):
    red = o1a_ref[...] + o1b_ref[...]
    wsum1 = red[:, 0:_H]
    sum16 = red[:, _H:_H + _DE]
    den1 = red[:, _H + _DE:_H + _DE + 1]
    deg = red[:, _H + _DE + 1:_H + _DE + 2]
    mean16 = sum16 / jnp.maximum(deg, 1.0)
    mean16_ref[...] = mean16
    has_deg = jnp.minimum(deg, 1.0)
    v12 = v12_ref[...]
    c12 = c12_ref[...]
    al_loop1 = (jnp.dot(mean16, v12[:, 0:1], preferred_element_type=jnp.float32)
                + c12[:, 0:1] * has_deg)
    asd = asd1v_ref[...]
    ex_l1 = jnp.exp(_leaky(asd[:, 0:1] + asd[:, 1:2] + al_loop1))
    xs1 = xs1_ref[...]
    h1 = (wsum1 + ex_l1 * xs1) / (den1 + ex_l1 + 1e-16) + b1_ref[...]
    h1 = jnp.maximum(h1, 0.0)
    xs2 = jnp.dot(h1, W2_ref[...], preferred_element_type=jnp.float32)
    xs2_ref[...] = xs2
    asd2 = jnp.dot(xs2, asd2w_ref[...], preferred_element_type=jnp.float32)
    asd2_ref[...] = asd2
    al_loop2 = (jnp.dot(mean16, v12[:, 1:2], preferred_element_type=jnp.float32)
                + c12[:, 1:2] * has_deg)
    exl2_ref[...] = jnp.exp(_leaky(asd2[:, 0:1] + asd2[:, 1:2] + al_loop2))


def _fold1(o1a, o1b, xs1, asd1v, v12, c12, W2, asd2w, b1):
    zero = lambda i: (0, 0)
    row = lambda i: (i, 0)
    return pl.pallas_call(
        _fold1_body,
        grid=(_N // _NBLK,),
        in_specs=[
            pl.BlockSpec((_NBLK, 96), row),
            pl.BlockSpec((_NBLK, 96), row),
            pl.BlockSpec((_NBLK, _H), row),
            pl.BlockSpec((_NBLK, 2), row),
            pl.BlockSpec((_DE, 2), zero),
            pl.BlockSpec((1, 2), zero),
            pl.BlockSpec((_H, _H), zero),
            pl.BlockSpec((_H, 2), zero),
            pl.BlockSpec((1, _H), zero),
        ],
        out_specs=(
            pl.BlockSpec((_NBLK, _H), row),
            pl.BlockSpec((_NBLK, 2), row),
            pl.BlockSpec((_NBLK, 1), row),
            pl.BlockSpec((_NBLK, _DE), row),
        ),
        out_shape=(
            jax.ShapeDtypeStruct((_N, _H), jnp.float32),
            jax.ShapeDtypeStruct((_N, 2), jnp.float32),
            jax.ShapeDtypeStruct((_N, 1), jnp.float32),
            jax.ShapeDtypeStruct((_N, _DE), jnp.float32),
        ),
    )(o1a, o1b, xs1, asd1v, v12, c12, W2, asd2w, b1)


def _gates_body(o2a_ref, o2b_ref, xs2_ref, exl2_ref, b2_ref,
                Wihi_ref, Wihf_ref, Wihg_ref, Wiho_ref,
                bgi_ref, bgf_ref, bgg_ref, bgo_ref,
                h2_ref, Xi_ref, Xf_ref, Xg_ref, Xo_ref):
    red = o2a_ref[...] + o2b_ref[...]
    exl2 = exl2_ref[...]
    xs2 = xs2_ref[...]
    h2 = ((red[:, 0:_H] + exl2 * xs2) /
          (red[:, _H:_H + 1] + exl2 + 1e-16) + b2_ref[...])
    h2_ref[...] = h2
    Xi_ref[...] = jnp.dot(h2, Wihi_ref[...], preferred_element_type=jnp.float32) + bgi_ref[...]
    Xf_ref[...] = jnp.dot(h2, Wihf_ref[...], preferred_element_type=jnp.float32) + bgf_ref[...]
    Xg_ref[...] = jnp.dot(h2, Wihg_ref[...], preferred_element_type=jnp.float32) + bgg_ref[...]
    Xo_ref[...] = jnp.dot(h2, Wiho_ref[...], preferred_element_type=jnp.float32) + bgo_ref[...]


def _gates_stage(o2a, o2b, xs2, exl2, b2, Wih4, bg4):
    zero = lambda i: (0, 0)
    row = lambda i: (i, 0)
    wspec = [pl.BlockSpec((_H, _H), zero)] * 4
    bspec = [pl.BlockSpec((1, _H), zero)] * 4
    nh = jax.ShapeDtypeStruct((_N, _H), jnp.float32)
    return pl.pallas_call(
        _gates_body,
        grid=(_N // _NBLK,),
        in_specs=[
            pl.BlockSpec((_NBLK, 80), row),
            pl.BlockSpec((_NBLK, 80), row),
            pl.BlockSpec((_NBLK, _H), row),
            pl.BlockSpec((_NBLK, 1), row),
            pl.BlockSpec((1, _H), zero),
        ] + wspec + bspec,
        out_specs=tuple([pl.BlockSpec((_NBLK, _H), row)] * 5),
        out_shape=(nh, nh, nh, nh, nh),
    )(o2a, o2b, xs2, exl2, b2, *Wih4, *bg4)


_LSTM_BLK = 8


def _lstm_body(Xi_ref, Xf_ref, Xg_ref, Xo_ref, Wi_ref, Wf_ref, Wg_ref,
               Wo_ref, hs_ref, hncn_ref):
    Wi = Wi_ref[...]
    Wf = Wf_ref[...]
    Wg = Wg_ref[...]
    Wo = Wo_ref[...]

    def blockstep(b, carry):
        h, c = carry
        sl = pl.ds(b * _LSTM_BLK, _LSTM_BLK)
        Xib = Xi_ref[sl, :]
        Xfb = Xf_ref[sl, :]
        Xgb = Xg_ref[sl, :]
        Xob = Xo_ref[sl, :]
        outs = []
        for j in range(_LSTM_BLK):
            jj = slice(j, j + 1)
            i = jax.nn.sigmoid(Xib[jj, :] + jnp.dot(h, Wi, preferred_element_type=jnp.float32))
            f = jax.nn.sigmoid(Xfb[jj, :] + jnp.dot(h, Wf, preferred_element_type=jnp.float32))
            gg = jnp.tanh(Xgb[jj, :] + jnp.dot(h, Wg, preferred_element_type=jnp.float32))
            o = jax.nn.sigmoid(Xob[jj, :] + jnp.dot(h, Wo, preferred_element_type=jnp.float32))
            c = f * c + i * gg
            h = o * jnp.tanh(c)
            outs.append(h)
        hs_ref[sl, :] = jnp.concatenate(outs, axis=0)
        return (h, c)

    h0 = jnp.zeros((1, _H), jnp.float32)
    c0 = jnp.zeros((1, _H), jnp.float32)
    h, c = lax.fori_loop(0, _N // _LSTM_BLK, blockstep, (h0, c0))
    hncn_ref[0:1, :] = h
    hncn_ref[1:2, :] = c


def _lstm(Xi, Xf, Xg, Xo, Whh4):
    return pl.pallas_call(
        _lstm_body,
        out_shape=(
            jax.ShapeDtypeStruct((_N, _H), jnp.float32),
            jax.ShapeDtypeStruct((2, _H), jnp.float32),
        ),
    )(Xi, Xf, Xg, Xo, *Whh4)


def _pq_body(h2_ref, hs_ref, Wa_ref, Wb_ref, Wc_ref, bs1_ref, P_ref, Q_ref):
    h2 = h2_ref[...]
    P_ref[...] = (jnp.dot(h2, Wa_ref[...], preferred_element_type=jnp.float32) +
                  jnp.dot(hs_ref[...], Wc_ref[...], preferred_element_type=jnp.float32) +
                  bs1_ref[...])
    Q_ref[...] = jnp.dot(h2, Wb_ref[...], preferred_element_type=jnp.float32)


def _pq_stage(h2, hs, Wa, Wb, Wc, bs1):
    zero = lambda i: (0, 0)
    row = lambda i: (i, 0)
    nh = jax.ShapeDtypeStruct((_N, _H), jnp.float32)
    return pl.pallas_call(
        _pq_body,
        grid=(_N // _NBLK,),
        in_specs=[
            pl.BlockSpec((_NBLK, _H), row),
            pl.BlockSpec((_NBLK, _H), row),
            pl.BlockSpec((_H, _H), zero),
            pl.BlockSpec((_H, _H), zero),
            pl.BlockSpec((_H, _H), zero),
            pl.BlockSpec((1, _H), zero),
        ],
        out_specs=(pl.BlockSpec((_NBLK, _H), row),
                   pl.BlockSpec((_NBLK, _H), row)),
        out_shape=(nh, nh),
    )(h2, hs, Wa, Wb, Wc, bs1)


def kernel(x, edge_index, edge_attr, W_ee, b_ee, W1, a_src1, a_dst1, a_edge1,
           We1, b1, W2, a_src2, a_dst2, a_edge2, We2, b2, W_ih, W_hh, b_ih,
           b_hh, Ws1, bs1, Ws2, bs2):
    src = edge_index[0]
    dst = edge_index[1]

    asd1 = jnp.stack([a_src1, a_dst1], axis=1)
    asd2w = jnp.stack([a_src2, a_dst2], axis=1)
    xs1, asd1v, v12, c12 = _proj1(
        x, W1, asd1, W_ee, b_ee.reshape(1, _H), We1,
        a_edge1.reshape(_H, 1), We2, a_edge2.reshape(_H, 1))
    ale12 = _edge_logits(edge_attr, v12, c12)

    o1 = _gat_pass1(src, dst, asd1v[:, 0], asd1v[:, 1], ale12[:, 0],
                    edge_attr, xs1)
    xs2, asd2v, exl2, _ = _fold1(o1[0, :_N], o1[1, :_N], xs1, asd1v, v12,
                                 c12, W2, asd2w, b1.reshape(1, _H))

    o2 = _gat_pass2(src, dst, asd2v[:, 0], asd2v[:, 1], ale12[:, 1],
                    edge_attr, xs2)

    WihT = W_ih.T
    WhhT = W_hh.T
    bg = b_ih + b_hh
    Wih4 = [WihT[:, k * _H:(k + 1) * _H] for k in range(4)]
    bg4 = [bg[k * _H:(k + 1) * _H].reshape(1, _H) for k in range(4)]
    Whh4 = [WhhT[:, k * _H:(k + 1) * _H] for k in range(4)]
    h2, Xi, Xf, Xg, Xo = _gates_stage(o2[0, :_N], o2[1, :_N], xs2, exl2,
                                      b2.reshape(1, _H), Wih4, bg4)
    hs, hncn = _lstm(Xi, Xf, Xg, Xo, Whh4)
    P, Q = _pq_stage(h2, hs, Ws1[0:_H], Ws1[_H:2 * _H], Ws1[2 * _H:3 * _H],
                     bs1.reshape(1, _H))

    scores = _scorer(src, dst, P, Q, Ws2[:, 0], jnp.broadcast_to(bs2, (16,)))

    return (scores, hncn[0].reshape(1, 1, _H), hncn[1].reshape(1, 1, _H))


# speedup vs baseline: 1.0027x; 1.0027x over previous
import functools

import jax
import jax.numpy as jnp
from jax import lax
from jax.experimental import pallas as pl
from jax.experimental.pallas import tpu as pltpu
from jax.experimental.pallas import tpu_sc as plsc

_N = 10000
_E = 320000
_H = 64
_DE = 16

_NC = 2
_NS = 16
_NW = _NC * _NS
_EW = _E // _NW
_B = 80
_NB = _EW // _B
_NPAD = 10240
_RPT = _NPAD // _NS

_mesh = plsc.VectorSubcoreMesh(core_axis_name="c", subcore_axis_name="s")


def _vbcast(v, idx):
    return lax.gather(
        v, idx[:, None],
        dimension_numbers=lax.GatherDimensionNumbers(
            offset_dims=(), collapsed_slice_dims=(0,), start_index_map=(0,)),
        slice_sizes=(1,),
        mode=lax.GatherScatterMode.PROMISE_IN_BOUNDS)


def _gat_pass_body(with_ea, W, src_hbm, dst_hbm, als_hbm, ald_hbm, ale_hbm,
                   ea_hbm, xs_hbm, out_hbm, acc, als_v, ald_v, srcv, dstv,
                   aev, eav, rows_v, payload_v, exv, sem):
    c = lax.axis_index("c")
    s = lax.axis_index("s")
    wid = s * _NC + c
    lane = lax.iota(jnp.int32, 16)
    zero16 = jnp.zeros((16,), jnp.float32)
    sel0 = jnp.where(lane == 0, 1.0, 0.0)
    sel1 = jnp.where(lane == 1, 1.0, 0.0)
    nch = W // 16

    pltpu.sync_copy(als_hbm, als_v)
    pltpu.sync_copy(ald_hbm, ald_v)

    def zrow(r, carry):
        for k in range(nch):
            payload_v[r, pl.ds(k * 16, 16)] = zero16
        return carry
    lax.fori_loop(0, _B, zrow, 0)

    base = s * _RPT
    for i in range(_RPT // _B):
        pltpu.sync_copy(payload_v, acc.at[pl.ds(base + i * _B, _B)])
    plsc.subcore_barrier()

    def block(b, carry):
        eb = wid * _EW + b * _B
        cps = [pltpu.async_copy(src_hbm.at[pl.ds(eb, _B)], srcv, sem),
               pltpu.async_copy(dst_hbm.at[pl.ds(eb, _B)], dstv, sem),
               pltpu.async_copy(ale_hbm.at[pl.ds(eb, _B)], aev, sem)]
        if with_ea:
            cps.append(pltpu.async_copy(ea_hbm.at[pl.ds(eb, _B)], eav, sem))
        for d in cps:
            d.wait()
        cp = pltpu.async_copy(xs_hbm.at[srcv], rows_v, sem)
        for g in range(_B // 16):
            s16 = srcv[pl.ds(g * 16, 16)]
            d16 = dstv[pl.ds(g * 16, 16)]
            a = (plsc.load_gather(als_v, [s16]) +
                 plsc.load_gather(ald_v, [d16]) +
                 aev[pl.ds(g * 16, 16)])
            a = jnp.maximum(a, a * 0.2)
            exv[pl.ds(g * 16, 16)] = jnp.exp(a)
        cp.wait()

        for g in range(_B // 16):
            exg = exv[pl.ds(g * 16, 16)]

            def row(r2, carry2):
                r = g * 16 + r2
                w = _vbcast(exg, jnp.full((16,), r2, jnp.int32))
                for k in range(4):
                    payload_v[r, pl.ds(k * 16, 16)] = (
                        rows_v[r, pl.ds(k * 16, 16)] * w)
                if with_ea:
                    payload_v[r, pl.ds(64, 16)] = eav[r, :]
                    payload_v[r, pl.ds(80, 16)] = w * sel0 + sel1
                else:
                    payload_v[r, pl.ds(64, 16)] = w * sel0
                return carry2
            lax.fori_loop(0, 16, row, 0)
        pltpu.sync_copy(payload_v, acc.at[dstv], add=True)
        return carry
    lax.fori_loop(0, _NB, block, 0)
    plsc.subcore_barrier()
    pltpu.sync_copy(acc.at[pl.ds(base, _RPT)], out_hbm.at[c, pl.ds(base, _RPT)])


def _make_gat_pass(with_ea):
    W = 96 if with_ea else 80
    return pl.kernel(
        functools.partial(_gat_pass_body, with_ea, W),
        out_type=jax.ShapeDtypeStruct((_NC, _NPAD, W), jnp.float32),
        mesh=_mesh,
        compiler_params=pltpu.CompilerParams(needs_layout_passes=False, use_tc_tiling_on_sc=False),
        scratch_types=[
            pltpu.VMEM_SHARED((_NPAD, W), jnp.float32),
            pltpu.VMEM((_N,), jnp.float32),
            pltpu.VMEM((_N,), jnp.float32),
            pltpu.VMEM((_B,), jnp.int32),
            pltpu.VMEM((_B,), jnp.int32),
            pltpu.VMEM((_B,), jnp.float32),
            pltpu.VMEM((_B, _DE), jnp.float32),
            pltpu.VMEM((_B, _H), jnp.float32),
            pltpu.VMEM((_B, W), jnp.float32),
            pltpu.VMEM((_B,), jnp.float32),
            pltpu.SemaphoreType.DMA,
        ],
    )


_gat_pass1 = _make_gat_pass(True)
_gat_pass2 = _make_gat_pass(False)


def _scorer_body(src_hbm, dst_hbm, p_hbm, q_hbm, ws2_hbm, bs2_hbm,
                 scores_hbm, srcv, dstv, prow, qrow, wsb, ws2v, bs2v,
                 scorev, semp, semq):
    c = lax.axis_index("c")
    s = lax.axis_index("s")
    wid = s * _NC + c
    lane = lax.iota(jnp.int32, 16)

    pltpu.sync_copy(ws2_hbm, ws2v)
    pltpu.sync_copy(bs2_hbm, bs2v)

    def bw(k, carry):
        colk = (jnp.full((16,), k, jnp.int32) + lane) & 63
        wsb[k, :] = plsc.load_gather(ws2v, [colk])
        return carry
    lax.fori_loop(0, _H, bw, 0)
    bias = bs2v[...]

    def block(b, carry):
        eb = wid * _EW + b * _B
        cp1 = pltpu.async_copy(src_hbm.at[pl.ds(eb, _B)], srcv, semp)
        cp2 = pltpu.async_copy(dst_hbm.at[pl.ds(eb, _B)], dstv, semp)
        cp1.wait()
        cp2.wait()
        cpp = pltpu.async_copy(p_hbm.at[srcv], prow, semp)
        cpq = pltpu.async_copy(q_hbm.at[dstv], qrow, semq)
        cpp.wait()
        cpq.wait()
        for g in range(_B // 16):
            rvec = g * 16 + lane
            acc = bias
            for k in range(_H):
                colk = (jnp.full((16,), k, jnp.int32) + lane) & 63
                pk = plsc.load_gather(prow, [rvec, colk])
                qk = plsc.load_gather(qrow, [rvec, colk])
                acc = acc + jnp.maximum(pk + qk, 0.0) * wsb[k, :]
            scorev[pl.ds(g * 16, 16)] = 1.0 / (1.0 + jnp.exp(-acc))
        pltpu.sync_copy(scorev, scores_hbm.at[pl.ds(eb, _B)])
        return carry
    lax.fori_loop(0, _NB, block, 0)


_scorer = pl.kernel(
    _scorer_body,
    out_type=jax.ShapeDtypeStruct((_E,), jnp.float32),
    mesh=_mesh,
    compiler_params=pltpu.CompilerParams(needs_layout_passes=False, use_tc_tiling_on_sc=False),
    scratch_types=[
        pltpu.VMEM((_B,), jnp.int32),
        pltpu.VMEM((_B,), jnp.int32),
        pltpu.VMEM((_B, _H), jnp.float32),
        pltpu.VMEM((_B, _H), jnp.float32),
        pltpu.VMEM((_H, 16), jnp.float32),
        pltpu.VMEM((_H,), jnp.float32),
        pltpu.VMEM((16,), jnp.float32),
        pltpu.VMEM((_B,), jnp.float32),
        pltpu.SemaphoreType.DMA,
        pltpu.SemaphoreType.DMA,
    ],
)


_EB = 8000


def _leaky(x):
    return jnp.maximum(x, 0.2 * x)


def _proj1_body(x_ref, W1_ref, asd1_ref, Wee_ref, bee_ref, We1_ref, ae1_ref,
                We2_ref, ae2_ref, xs_ref, asd_ref, v12_ref, c12_ref):
    w1e = jnp.dot(We1_ref[...], ae1_ref[...], preferred_element_type=jnp.float32)
    w2e = jnp.dot(We2_ref[...], ae2_ref[...], preferred_element_type=jnp.float32)
    wboth = jnp.concatenate([w1e, w2e], axis=1)
    v12_ref[...] = jnp.dot(Wee_ref[...], wboth, preferred_element_type=jnp.float32)
    c12_ref[...] = jnp.dot(bee_ref[...], wboth, preferred_element_type=jnp.float32)
    xs = jnp.dot(x_ref[...], W1_ref[...], preferred_element_type=jnp.float32)
    xs_ref[...] = xs
    asd_ref[...] = jnp.dot(xs, asd1_ref[...], preferred_element_type=jnp.float32)


_NBLK = 1000


def _proj1(x, W1, asd1, Wee, bee, We1, ae1, We2, ae2):
    zero = lambda i: (0, 0)
    return pl.pallas_call(
        _proj1_body,
        grid=(_N // _NBLK,),
        in_specs=[
            pl.BlockSpec((_NBLK, 128), lambda i: (i, 0)),
            pl.BlockSpec((128, _H), zero),
            pl.BlockSpec((_H, 2), zero),
            pl.BlockSpec((_DE, _H), zero),
            pl.BlockSpec((1, _H), zero),
            pl.BlockSpec((_H, _H), zero),
            pl.BlockSpec((_H, 1), zero),
            pl.BlockSpec((_H, _H), zero),
            pl.BlockSpec((_H, 1), zero),
        ],
        out_specs=(
            pl.BlockSpec((_NBLK, _H), lambda i: (i, 0)),
            pl.BlockSpec((_NBLK, 2), lambda i: (i, 0)),
            pl.BlockSpec((_DE, 2), zero),
            pl.BlockSpec((1, 2), zero),
        ),
        out_shape=(
            jax.ShapeDtypeStruct((_N, _H), jnp.float32),
            jax.ShapeDtypeStruct((_N, 2), jnp.float32),
            jax.ShapeDtypeStruct((_DE, 2), jnp.float32),
            jax.ShapeDtypeStruct((1, 2), jnp.float32),
        ),
    )(x, W1, asd1, Wee, bee, We1, ae1, We2, ae2)


def _elog_body(ea_ref, v_ref, c_ref, out_ref):
    out_ref[...] = jnp.dot(ea_ref[...], v_ref[...],
                           preferred_element_type=jnp.float32) + c_ref[...]


def _edge_logits(edge_attr, v12, c12):
    return pl.pallas_call(
        _elog_body,
        grid=(_E // _EB,),
        in_specs=[
            pl.BlockSpec((_EB, _DE), lambda i: (i, 0)),
            pl.BlockSpec((_DE, 2), lambda i: (0, 0)),
            pl.BlockSpec((1, 2), lambda i: (0, 0)),
        ],
        out_specs=pl.BlockSpec((_EB, 2), lambda i: (i, 0)),
        out_shape=jax.ShapeDtypeStruct((_E, 2), jnp.float32),
    )(edge_attr, v12, c12)


def _fold1_body(o1a_ref, o1b_ref, xs1_ref, asd1v_ref, v12_ref, c12_ref,
                W2_ref, asd2w_ref, b1_ref, xs2_ref, asd2_ref, exl2_ref,
                mean16_ref):
    red = o1a_ref[...] + o1b_ref[...]
    wsum1 = red[:, 0:_H]
    sum16 = red[:, _H:_H + _DE]
    den1 = red[:, _H + _DE:_H + _DE + 1]
    deg = red[:, _H + _DE + 1:_H + _DE + 2]
    mean16 = sum16 / jnp.maximum(deg, 1.0)
    mean16_ref[...] = mean16
    has_deg = jnp.minimum(deg, 1.0)
    v12 = v12_ref[...]
    c12 = c12_ref[...]
    al_loop1 = (jnp.dot(mean16, v12[:, 0:1], preferred_element_type=jnp.float32)
                + c12[:, 0:1] * has_deg)
    asd = asd1v_ref[...]
    ex_l1 = jnp.exp(_leaky(asd[:, 0:1] + asd[:, 1:2] + al_loop1))
    xs1 = xs1_ref[...]
    h1 = (wsum1 + ex_l1 * xs1) / (den1 + ex_l1 + 1e-16) + b1_ref[...]
    h1 = jnp.maximum(h1, 0.0)
    xs2 = jnp.dot(h1, W2_ref[...], preferred_element_type=jnp.float32)
    xs2_ref[...] = xs2
    asd2 = jnp.dot(xs2, asd2w_ref[...], preferred_element_type=jnp.float32)
    asd2_ref[...] = asd2
    al_loop2 = (jnp.dot(mean16, v12[:, 1:2], preferred_element_type=jnp.float32)
                + c12[:, 1:2] * has_deg)
    exl2_ref[...] = jnp.exp(_leaky(asd2[:, 0:1] + asd2[:, 1:2] + al_loop2))


def _fold1(o1a, o1b, xs1, asd1v, v12, c12, W2, asd2w, b1):
    zero = lambda i: (0, 0)
    row = lambda i: (i, 0)
    return pl.pallas_call(
        _fold1_body,
        grid=(_N // _NBLK,),
        in_specs=[
            pl.BlockSpec((_NBLK, 96), row),
            pl.BlockSpec((_NBLK, 96), row),
            pl.BlockSpec((_NBLK, _H), row),
            pl.BlockSpec((_NBLK, 2), row),
            pl.BlockSpec((_DE, 2), zero),
            pl.BlockSpec((1, 2), zero),
            pl.BlockSpec((_H, _H), zero),
            pl.BlockSpec((_H, 2), zero),
            pl.BlockSpec((1, _H), zero),
        ],
        out_specs=(
            pl.BlockSpec((_NBLK, _H), row),
            pl.BlockSpec((_NBLK, 2), row),
            pl.BlockSpec((_NBLK, 1), row),
            pl.BlockSpec((_NBLK, _DE), row),
        ),
        out_shape=(
            jax.ShapeDtypeStruct((_N, _H), jnp.float32),
            jax.ShapeDtypeStruct((_N, 2), jnp.float32),
            jax.ShapeDtypeStruct((_N, 1), jnp.float32),
            jax.ShapeDtypeStruct((_N, _DE), jnp.float32),
        ),
    )(o1a, o1b, xs1, asd1v, v12, c12, W2, asd2w, b1)


def _gates_body(o2a_ref, o2b_ref, xs2_ref, exl2_ref, b2_ref,
                Wihi_ref, Wihf_ref, Wihg_ref, Wiho_ref,
                bgi_ref, bgf_ref, bgg_ref, bgo_ref,
                h2_ref, Xi_ref, Xf_ref, Xg_ref, Xo_ref):
    red = o2a_ref[...] + o2b_ref[...]
    exl2 = exl2_ref[...]
    xs2 = xs2_ref[...]
    h2 = ((red[:, 0:_H] + exl2 * xs2) /
          (red[:, _H:_H + 1] + exl2 + 1e-16) + b2_ref[...])
    h2_ref[...] = h2
    Xi_ref[...] = jnp.dot(h2, Wihi_ref[...], preferred_element_type=jnp.float32) + bgi_ref[...]
    Xf_ref[...] = jnp.dot(h2, Wihf_ref[...], preferred_element_type=jnp.float32) + bgf_ref[...]
    Xg_ref[...] = jnp.dot(h2, Wihg_ref[...], preferred_element_type=jnp.float32) + bgg_ref[...]
    Xo_ref[...] = jnp.dot(h2, Wiho_ref[...], preferred_element_type=jnp.float32) + bgo_ref[...]


def _gates_stage(o2a, o2b, xs2, exl2, b2, Wih4, bg4):
    zero = lambda i: (0, 0)
    row = lambda i: (i, 0)
    wspec = [pl.BlockSpec((_H, _H), zero)] * 4
    bspec = [pl.BlockSpec((1, _H), zero)] * 4
    nh = jax.ShapeDtypeStruct((_N, _H), jnp.float32)
    return pl.pallas_call(
        _gates_body,
        grid=(_N // _NBLK,),
        in_specs=[
            pl.BlockSpec((_NBLK, 80), row),
            pl.BlockSpec((_NBLK, 80), row),
            pl.BlockSpec((_NBLK, _H), row),
            pl.BlockSpec((_NBLK, 1), row),
            pl.BlockSpec((1, _H), zero),
        ] + wspec + bspec,
        out_specs=tuple([pl.BlockSpec((_NBLK, _H), row)] * 5),
        out_shape=(nh, nh, nh, nh, nh),
    )(o2a, o2b, xs2, exl2, b2, *Wih4, *bg4)


_LSTM_BLK = 8


def _lstm_body(Xi_ref, Xf_ref, Xg_ref, Xo_ref, h2_ref, Wi_ref, Wf_ref,
               Wg_ref, Wo_ref, Wa_ref, Wb_ref, Wc_ref, bs1_ref,
               P_ref, Q_ref, hncn_ref, hs_ref):
    Wi = Wi_ref[...]
    Wf = Wf_ref[...]
    Wg = Wg_ref[...]
    Wo = Wo_ref[...]

    def blockstep(b, carry):
        h, c = carry
        sl = pl.ds(b * _LSTM_BLK, _LSTM_BLK)
        Xib = Xi_ref[sl, :]
        Xfb = Xf_ref[sl, :]
        Xgb = Xg_ref[sl, :]
        Xob = Xo_ref[sl, :]
        outs = []
        for j in range(_LSTM_BLK):
            jj = slice(j, j + 1)
            i = jax.nn.sigmoid(Xib[jj, :] + jnp.dot(h, Wi, preferred_element_type=jnp.float32))
            f = jax.nn.sigmoid(Xfb[jj, :] + jnp.dot(h, Wf, preferred_element_type=jnp.float32))
            gg = jnp.tanh(Xgb[jj, :] + jnp.dot(h, Wg, preferred_element_type=jnp.float32))
            o = jax.nn.sigmoid(Xob[jj, :] + jnp.dot(h, Wo, preferred_element_type=jnp.float32))
            c = f * c + i * gg
            h = o * jnp.tanh(c)
            outs.append(h)
        hs_ref[sl, :] = jnp.concatenate(outs, axis=0)
        return (h, c)

    h0 = jnp.zeros((1, _H), jnp.float32)
    c0 = jnp.zeros((1, _H), jnp.float32)
    h, c = lax.fori_loop(0, _N // _LSTM_BLK, blockstep, (h0, c0))
    hncn_ref[0:1, :] = h
    hncn_ref[1:2, :] = c
    hs = hs_ref[...]
    h2 = h2_ref[...]
    P_ref[...] = (jnp.dot(h2, Wa_ref[...], preferred_element_type=jnp.float32) +
                  jnp.dot(hs, Wc_ref[...], preferred_element_type=jnp.float32) +
                  bs1_ref[...])
    Q_ref[...] = jnp.dot(h2, Wb_ref[...], preferred_element_type=jnp.float32)


def _lstm_pq(Xi, Xf, Xg, Xo, h2, Whh4, Wa, Wb, Wc, bs1):
    nh = jax.ShapeDtypeStruct((_N, _H), jnp.float32)
    return pl.pallas_call(
        _lstm_body,
        out_shape=(
            nh, nh,
            jax.ShapeDtypeStruct((2, _H), jnp.float32),
        ),
        scratch_shapes=[pltpu.VMEM((_N, _H), jnp.float32)],
    )(Xi, Xf, Xg, Xo, h2, *Whh4, Wa, Wb, Wc, bs1)


def kernel(x, edge_index, edge_attr, W_ee, b_ee, W1, a_src1, a_dst1, a_edge1,
           We1, b1, W2, a_src2, a_dst2, a_edge2, We2, b2, W_ih, W_hh, b_ih,
           b_hh, Ws1, bs1, Ws2, bs2):
    src = edge_index[0]
    dst = edge_index[1]

    asd1 = jnp.stack([a_src1, a_dst1], axis=1)
    asd2w = jnp.stack([a_src2, a_dst2], axis=1)
    xs1, asd1v, v12, c12 = _proj1(
        x, W1, asd1, W_ee, b_ee.reshape(1, _H), We1,
        a_edge1.reshape(_H, 1), We2, a_edge2.reshape(_H, 1))
    ale12 = _edge_logits(edge_attr, v12, c12)

    o1 = _gat_pass1(src, dst, asd1v[:, 0], asd1v[:, 1], ale12[:, 0],
                    edge_attr, xs1)
    xs2, asd2v, exl2, _ = _fold1(o1[0, :_N], o1[1, :_N], xs1, asd1v, v12,
                                 c12, W2, asd2w, b1.reshape(1, _H))

    o2 = _gat_pass2(src, dst, asd2v[:, 0], asd2v[:, 1], ale12[:, 1],
                    edge_attr, xs2)

    WihT = W_ih.T
    WhhT = W_hh.T
    bg = b_ih + b_hh
    Wih4 = [WihT[:, k * _H:(k + 1) * _H] for k in range(4)]
    bg4 = [bg[k * _H:(k + 1) * _H].reshape(1, _H) for k in range(4)]
    Whh4 = [WhhT[:, k * _H:(k + 1) * _H] for k in range(4)]
    h2, Xi, Xf, Xg, Xo = _gates_stage(o2[0, :_N], o2[1, :_N], xs2, exl2,
                                      b2.reshape(1, _H), Wih4, bg4)
    P, Q, hncn = _lstm_pq(Xi, Xf, Xg, Xo, h2, Whh4, Ws1[0:_H],
                          Ws1[_H:2 * _H], Ws1[2 * _H:3 * _H],
                          bs1.reshape(1, _H))

    scores = _scorer(src, dst, P, Q, Ws2[:, 0], jnp.broadcast_to(bs2, (16,)))

    return (scores, hncn[0].reshape(1, 1, _H), hncn[1].reshape(1, 1, _H))
